# baseline (speedup 1.0000x reference)
import jax
import jax.numpy as jnp
from jax.experimental import pallas as pl
from jax.experimental.pallas import tpu as pltpu

OUT_DIM = 32
IN_DIM = 50000
BATCH = 4096
_TAU = 10.0 * (0.01 / 10.0) ** (1.0 / 10000.0)

KB = 1000
KSTEPS = IN_DIM // KB


def _fused_kernel(xt_ref, at_ref, gt_ref, out_ref, acc_ref, ssum_ref):
    k = pl.program_id(0)
    e_t = jnp.exp((at_ref[...] + gt_ref[...]) * (1.0 / _TAU))
    e = e_t.T
    part = jnp.dot(e, xt_ref[...], preferred_element_type=jnp.float32)
    s = jnp.sum(e_t, axis=0, keepdims=True)

    @pl.when(k == 0)
    def _init():
        acc_ref[...] = part
        ssum_ref[...] = s

    @pl.when(k != 0)
    def _acc():
        acc_ref[...] += part
        ssum_ref[...] += s

    @pl.when(k == KSTEPS - 1)
    def _fin():
        out_ref[...] = acc_ref[...] * (1.0 / ssum_ref[...]).T


def kernel(x, alphas, gumbel):
    xt = jnp.transpose(x)
    at = jnp.transpose(alphas)
    gt = jnp.transpose(gumbel)

    out_t = pl.pallas_call(
        _fused_kernel,
        grid=(KSTEPS,),
        in_specs=[
            pl.BlockSpec((KB, BATCH), lambda k: (k, 0)),
            pl.BlockSpec((KB, OUT_DIM), lambda k: (k, 0)),
            pl.BlockSpec((KB, OUT_DIM), lambda k: (k, 0)),
        ],
        out_specs=pl.BlockSpec((OUT_DIM, BATCH), lambda k: (0, 0)),
        out_shape=jax.ShapeDtypeStruct((OUT_DIM, BATCH), jnp.float32),
        scratch_shapes=[
            pltpu.VMEM((OUT_DIM, BATCH), jnp.float32),
            pltpu.VMEM((1, OUT_DIM), jnp.float32),
        ],
    )(xt, at, gt)
    return (jnp.transpose(out_t), None)


# speedup vs baseline: 1.0990x; 1.0990x over previous
import jax
import jax.numpy as jnp
from jax.experimental import pallas as pl
from jax.experimental.pallas import tpu as pltpu

OUT_DIM = 32
IN_DIM = 50000
BATCH = 4096
_TAU = 10.0 * (0.01 / 10.0) ** (1.0 / 10000.0)

KB = 1000
KSTEPS = IN_DIM // KB


def _softmax_t_kernel(a_ref, g_ref, out_ref):
    logits = (a_ref[...] + g_ref[...]) * (1.0 / _TAU)
    m = jnp.max(logits, axis=-1, keepdims=True)
    e = jnp.exp(logits - m)
    s = jnp.sum(e, axis=-1, keepdims=True)
    out_ref[...] = (e / s).T


def _matmul_kernel(xt_ref, mt_ref, out_ref, acc_ref):
    k = pl.program_id(0)
    m = mt_ref[...].T
    part = jnp.dot(m, xt_ref[...], preferred_element_type=jnp.float32)

    @pl.when(k == 0)
    def _init():
        acc_ref[...] = part

    @pl.when(k != 0)
    def _acc():
        acc_ref[...] += part

    @pl.when(k == KSTEPS - 1)
    def _fin():
        out_ref[...] = acc_ref[...]


def kernel(x, alphas, gumbel):
    xt = jnp.transpose(x)

    mask_t = pl.pallas_call(
        _softmax_t_kernel,
        out_shape=jax.ShapeDtypeStruct((IN_DIM, OUT_DIM), jnp.float32),
    )(alphas, gumbel)

    out_t = pl.pallas_call(
        _matmul_kernel,
        grid=(KSTEPS,),
        in_specs=[
            pl.BlockSpec((KB, BATCH), lambda k: (k, 0)),
            pl.BlockSpec((KB, OUT_DIM), lambda k: (k, 0)),
        ],
        out_specs=pl.BlockSpec((OUT_DIM, BATCH), lambda k: (0, 0)),
        out_shape=jax.ShapeDtypeStruct((OUT_DIM, BATCH), jnp.float32),
        scratch_shapes=[pltpu.VMEM((OUT_DIM, BATCH), jnp.float32)],
    )(xt, mask_t)
    return (jnp.transpose(out_t), None)
